# baseline (speedup 1.0000x reference)
import functools

import jax
import jax.numpy as jnp
from jax import lax
from jax.experimental import pallas as pl
from jax.experimental.pallas import tpu as pltpu
from jax.experimental.pallas import tpu_sc as plsc

NC = 2
NS = 16
NW = NC * NS

S = 10
D = 128
LANES = 16
NDV = D // LANES

C = 16
PR = 512


def _prep_body(sn_ref, nd_ref, cf_ref, snp_ref, cfp_ref):
    sn = sn_ref[...]
    nd = nd_ref[...]
    cf = cf_ref[...]
    fill = jnp.full((PR, LANES - 1 - S), -1, jnp.int32)
    snp_ref[...] = jnp.concatenate([sn, fill, nd[:, None]], axis=1)
    cfp_ref[...] = jnp.concatenate(
        [cf, jnp.zeros((PR, LANES - S), jnp.float32)], axis=1)


def _sc_body(snp_hbm, cfp_hbm, table_hbm, eps_hbm, out_hbm,
             sn_v, cf_v, eps_v, emb_v, out_v, sem_a, sem_b, sem_o):
    wid = lax.axis_index("s") * NC + lax.axis_index("c")
    rows_per_w = snp_hbm.shape[0] // NW
    nch = rows_per_w // C
    wrb = wid * rows_per_w

    pltpu.sync_copy(eps_hbm, eps_v)
    eps_vec = eps_v[...]
    lane = lax.iota(jnp.int32, LANES)
    valid = lane < S

    def fire_a(c, a):
        rb = wrb + c * C
        pltpu.async_copy(snp_hbm.at[pl.ds(rb, C)], sn_v.at[a], sem_a.at[a])
        pltpu.async_copy(cfp_hbm.at[pl.ds(rb, C)], cf_v.at[a], sem_a.at[a])

    def wait_a(a):
        pltpu.make_async_copy(snp_hbm.at[pl.ds(0, C)], sn_v.at[a],
                              sem_a.at[a]).wait()
        pltpu.make_async_copy(cfp_hbm.at[pl.ds(0, C)], cf_v.at[a],
                              sem_a.at[a]).wait()

    def fire_b(a, p):
        for r in range(C):
            pltpu.async_copy(table_hbm.at[sn_v.at[a, r, pl.ds(0, S)]],
                             emb_v.at[p, r], sem_b.at[p])

    def wait_b(p):
        pltpu.make_async_copy(table_hbm.at[pl.ds(0, C * S)], emb_v.at[p],
                              sem_b.at[p]).wait()

    def fire_o(c, p):
        pltpu.async_copy(out_v.at[p], out_hbm.at[pl.ds(wrb + c * C, C)],
                         sem_o.at[p])

    def wait_o(p):
        pltpu.make_async_copy(out_v.at[p], out_hbm.at[pl.ds(0, C)],
                              sem_o.at[p]).wait()

    def compute(a, p):
        def row_body(i, carry):
            sn_vec = sn_v[a, i]
            cf_vec = cf_v[a, i]
            node = sn_vec[LANES - 1]
            mbool = valid & (sn_vec == node)
            mfv = jnp.where(mbool, jnp.float32(1.0), jnp.float32(0.0))
            wu_vec = cf_vec * (jnp.float32(1.0) - mfv)
            wv_vec = cf_vec * mfv
            acc_u = [jnp.zeros((LANES,), jnp.float32) for _ in range(NDV)]
            acc_v = [jnp.zeros((LANES,), jnp.float32) for _ in range(NDV)]
            cnt = jnp.float32(0.0)
            for j in range(S):
                wu = wu_vec[j]
                wv = wv_vec[j]
                cnt = cnt + mfv[j]
                for d in range(NDV):
                    e = emb_v[p, i, j, pl.ds(d * LANES, LANES)]
                    acc_u[d] = acc_u[d] + wu * e
                    acc_v[d] = acc_v[d] + wv * e
            denom = jnp.full((LANES,), jnp.float32(S), jnp.float32) - cnt
            inv = jnp.full((LANES,), jnp.float32(1.0), jnp.float32) / denom
            for d in range(NDV):
                out_v[p, i, pl.ds(d * LANES, LANES)] = acc_u[d] * inv
                out_v[p, i, pl.ds(D + d * LANES, LANES)] = acc_v[d] * eps_vec
            return carry

        lax.fori_loop(0, C, row_body, 0)

    fire_a(0, 0)
    fire_a(1, 1)
    wait_a(0)
    fire_b(0, 0)

    def chunk_body(c, carry):
        p = c & 1
        a = c % 3

        @pl.when(c + 1 < nch)
        def _():
            wait_a((c + 1) % 3)
            fire_b((c + 1) % 3, 1 - p)

        wait_b(p)

        @pl.when(c + 2 < nch)
        def _():
            fire_a(c + 2, (c + 2) % 3)

        @pl.when(c >= 2)
        def _():
            wait_o(p)

        compute(a, p)
        fire_o(c, p)
        return carry

    lax.fori_loop(0, nch, chunk_body, 0)
    wait_o(0)
    wait_o(1)


def kernel(nodes, samp_neighs, structural_coeff, features_table, eps):
    B, s = samp_neighs.shape
    N, d = features_table.shape
    assert s == S and d == D

    sn32 = samp_neighs.astype(jnp.int32)
    nodes32 = nodes.astype(jnp.int32)
    eps16 = jnp.full((LANES,), eps[0], dtype=jnp.float32)

    sn_pad, cf_pad = pl.pallas_call(
        _prep_body,
        grid=(B // PR,),
        in_specs=[
            pl.BlockSpec((PR, S), lambda i: (i, 0)),
            pl.BlockSpec((PR,), lambda i: (i,)),
            pl.BlockSpec((PR, S), lambda i: (i, 0)),
        ],
        out_specs=[
            pl.BlockSpec((PR, LANES), lambda i: (i, 0)),
            pl.BlockSpec((PR, LANES), lambda i: (i, 0)),
        ],
        out_shape=[
            jax.ShapeDtypeStruct((B, LANES), jnp.int32),
            jax.ShapeDtypeStruct((B, LANES), jnp.float32),
        ],
    )(sn32, nodes32, structural_coeff)

    mesh = plsc.VectorSubcoreMesh(core_axis_name="c", subcore_axis_name="s",
                                  num_cores=NC, num_subcores=NS)
    run = functools.partial(
        pl.kernel,
        out_type=jax.ShapeDtypeStruct((B, 2 * D), jnp.float32),
        mesh=mesh,
        scratch_types=[
            pltpu.VMEM((3, C, LANES), jnp.int32),
            pltpu.VMEM((3, C, LANES), jnp.float32),
            pltpu.VMEM((LANES,), jnp.float32),
            pltpu.VMEM((2, C, S, D), jnp.float32),
            pltpu.VMEM((2, C, 2 * D), jnp.float32),
            pltpu.SemaphoreType.DMA((3,)),
            pltpu.SemaphoreType.DMA((2,)),
            pltpu.SemaphoreType.DMA((2,)),
        ],
    )(_sc_body)
    return run(sn_pad, cf_pad, features_table, eps16)


# speedup vs baseline: 1.1578x; 1.1578x over previous
import functools

import jax
import jax.numpy as jnp
from jax import lax
from jax.experimental import pallas as pl
from jax.experimental.pallas import tpu as pltpu
from jax.experimental.pallas import tpu_sc as plsc

NC = 2
NS = 16
NW = NC * NS

S = 10
D = 128
LANES = 16
NDV = D // LANES

C = 32
CS = C * S
IDX_W = 80
GATHERS = CS // IDX_W


def _splat(x):
    return jnp.full((LANES,), x, jnp.int32)


def _body(nodes_hbm, sn_hbm, cf_hbm, table_hbm, eps_hbm, out_hbm,
          nd_v, sn_v, cf_v, eps_v, emb_v, out_v, sem_a, sem_b, sem_o):
    wid = lax.axis_index("s") * NC + lax.axis_index("c")
    rows_per_w = nodes_hbm.shape[0] // NW
    nch = rows_per_w // C
    wrb = wid * rows_per_w

    pltpu.sync_copy(eps_hbm, eps_v)
    eps_vec = eps_v[...]
    lane = lax.iota(jnp.int32, LANES)
    valid = lane < S
    col_ids = jnp.minimum(lane, S - 1)

    def fire_a(c, a):
        rb = wrb + c * C
        pltpu.async_copy(nodes_hbm.at[pl.ds(rb, C)],
                         nd_v.at[pl.ds(a * C, C)], sem_a.at[a])
        pltpu.async_copy(sn_hbm.at[pl.ds(rb * S, CS)],
                         sn_v.at[pl.ds(a * CS, CS)], sem_a.at[a])
        pltpu.async_copy(cf_hbm.at[pl.ds(rb, C)], cf_v.at[a], sem_a.at[a])

    def wait_a(a):
        pltpu.make_async_copy(nodes_hbm.at[pl.ds(0, C)],
                              nd_v.at[pl.ds(a * C, C)], sem_a.at[a]).wait()
        pltpu.make_async_copy(sn_hbm.at[pl.ds(0, CS)],
                              sn_v.at[pl.ds(a * CS, CS)], sem_a.at[a]).wait()
        pltpu.make_async_copy(cf_hbm.at[pl.ds(0, C)], cf_v.at[a],
                              sem_a.at[a]).wait()

    def fire_b(a, p):
        for g in range(GATHERS):
            pltpu.async_copy(
                table_hbm.at[sn_v.at[pl.ds(a * CS + g * IDX_W, IDX_W)]],
                emb_v.at[p, pl.ds(g * IDX_W, IDX_W)], sem_b.at[p])

    def wait_b(p):
        pltpu.make_async_copy(table_hbm.at[pl.ds(0, CS)], emb_v.at[p],
                              sem_b.at[p]).wait()

    def fire_o(c, p):
        pltpu.async_copy(out_v.at[p], out_hbm.at[pl.ds(wrb + c * C, C)],
                         sem_o.at[p])

    def wait_o(p):
        pltpu.make_async_copy(out_v.at[p], out_hbm.at[pl.ds(0, C)],
                              sem_o.at[p]).wait()

    def compute(a, p):
        def row_body(i, carry):
            base = i * S
            sbase = a * CS + base
            sn_vec = sn_v[pl.ds(sbase, LANES)]
            cf_vec = cf_v[a, i]
            node = nd_v[pl.ds(a * C + i, LANES)][0]
            mbool = valid & (sn_vec == node)
            mfv = jnp.where(mbool, jnp.float32(1.0), jnp.float32(0.0))
            wu_vec = cf_vec * (jnp.float32(1.0) - mfv)
            wv_vec = cf_vec * mfv
            acc_u = [jnp.zeros((LANES,), jnp.float32) for _ in range(NDV)]
            acc_v = [jnp.zeros((LANES,), jnp.float32) for _ in range(NDV)]
            cnt = jnp.float32(0.0)
            for j in range(S):
                wu = wu_vec[j]
                wv = wv_vec[j]
                cnt = cnt + mfv[j]
                for d in range(NDV):
                    e = emb_v[p, base + j, pl.ds(d * LANES, LANES)]
                    acc_u[d] = acc_u[d] + wu * e
                    acc_v[d] = acc_v[d] + wv * e
            denom = jnp.full((LANES,), jnp.float32(S), jnp.float32) - cnt
            inv = jnp.full((LANES,), jnp.float32(1.0), jnp.float32) / denom
            for d in range(NDV):
                out_v[p, i, pl.ds(d * LANES, LANES)] = acc_u[d] * inv
                out_v[p, i, pl.ds(D + d * LANES, LANES)] = acc_v[d] * eps_vec
            return carry

        lax.fori_loop(0, C, row_body, 0)

    fire_a(0, 0)
    fire_a(1, 1)
    wait_a(0)
    fire_b(0, 0)

    def chunk_body(c, carry):
        p = c & 1
        a = c % 3

        @pl.when(c + 1 < nch)
        def _():
            wait_a((c + 1) % 3)
            fire_b((c + 1) % 3, 1 - p)

        wait_b(p)

        @pl.when(c + 2 < nch)
        def _():
            fire_a(c + 2, (c + 2) % 3)

        @pl.when(c >= 2)
        def _():
            wait_o(p)

        compute(a, p)
        fire_o(c, p)
        return carry

    lax.fori_loop(0, nch, chunk_body, 0)
    wait_o(0)
    wait_o(1)


def kernel(nodes, samp_neighs, structural_coeff, features_table, eps):
    B, s = samp_neighs.shape
    N, d = features_table.shape
    assert s == S and d == D

    sn_flat = samp_neighs.astype(jnp.int32).reshape(-1)
    cf_flat = jnp.pad(structural_coeff, ((0, 0), (0, LANES - S)))
    nodes32 = nodes.astype(jnp.int32)
    eps16 = jnp.full((LANES,), eps[0], dtype=jnp.float32)

    mesh = plsc.VectorSubcoreMesh(core_axis_name="c", subcore_axis_name="s",
                                  num_cores=NC, num_subcores=NS)
    run = functools.partial(
        pl.kernel,
        out_type=jax.ShapeDtypeStruct((B, 2 * D), jnp.float32),
        mesh=mesh,
        scratch_types=[
            pltpu.VMEM((3 * C + LANES,), jnp.int32),
            pltpu.VMEM((3 * CS + LANES,), jnp.int32),
            pltpu.VMEM((3, C, LANES), jnp.float32),
            pltpu.VMEM((LANES,), jnp.float32),
            pltpu.VMEM((2, CS, D), jnp.float32),
            pltpu.VMEM((2, C, 2 * D), jnp.float32),
            pltpu.SemaphoreType.DMA((3,)),
            pltpu.SemaphoreType.DMA((2,)),
            pltpu.SemaphoreType.DMA((2,)),
        ],
    )(_body)
    return run(nodes32, sn_flat, cf_flat, features_table, eps16)
